# Initial kernel scaffold; baseline (speedup 1.0000x reference)
import jax
import jax.numpy as jnp
from jax.experimental import pallas as pl


def kernel(logits, top_ps, min_ps, top_ks, sampling_seed, positions):
    raise NotImplementedError("write your pallas kernel here")


# speedup vs baseline: 43.0906x; 43.0906x over previous
import functools

import jax
import jax.numpy as jnp
from jax import lax
from jax.experimental import pallas as pl
from jax.experimental.pallas import tpu as pltpu
from jax.experimental.pallas import tpu_sc as plsc

B = 128
V = 100000
CAP = 2048
KMAX = 1024
EPS = 1e-10
ROW_BLK = 16
N_BLK = B // ROW_BLK


def _k1_body(x_ref, m_ref, z_ref, thr_ref):
    x = x_ref[...]
    m = jnp.max(x, axis=1, keepdims=True)
    z = jnp.sum(jnp.exp(x - m), axis=1, keepdims=True)

    i = lax.bitcast_convert_type(x, jnp.int32)
    u = i.astype(jnp.uint32)
    u = jnp.where(i >= 0, u + jnp.uint32(0x80000000), ~u)

    def bit_step(k, t):
        bit = (jnp.uint32(31) - k.astype(jnp.uint32))
        cand = t | (jnp.uint32(1) << bit)
        cnt = jnp.sum((u >= cand).astype(jnp.int32), axis=1, keepdims=True)
        return jnp.where(cnt >= KMAX, cand, t)

    t = lax.fori_loop(0, 32, bit_step, jnp.zeros((ROW_BLK, 1), jnp.uint32))
    ti = jnp.where(t >= jnp.uint32(0x80000000), t - jnp.uint32(0x80000000), ~t)
    thr = lax.bitcast_convert_type(ti, jnp.float32)

    m_ref[...] = m
    z_ref[...] = z
    thr_ref[...] = thr


def _k1_call(logits):
    out = jax.ShapeDtypeStruct((B, 1), jnp.float32)
    return pl.pallas_call(
        _k1_body,
        grid=(N_BLK,),
        in_specs=[pl.BlockSpec((ROW_BLK, V), lambda i: (i, 0))],
        out_specs=[pl.BlockSpec((ROW_BLK, 1), lambda i: (i, 0))] * 3,
        out_shape=[out, out, out],
    )(logits)


def _k2_body(logits_hbm, thr_hbm, cv_hbm, ci_hbm, cnt_hbm,
             xbuf, cvbuf, cibuf, tbuf, cntbuf):
    c = lax.axis_index("c")
    s = lax.axis_index("s")
    wid = s * 2 + c
    for rr in range(4):
        r = wid * 4 + rr
        pltpu.sync_copy(thr_hbm.at[r], tbuf)
        pltpu.sync_copy(logits_hbm.at[r], xbuf)
        tv = tbuf[...]

        def chunk(j, off):
            v = xbuf[pl.ds(j * 16, 16)]
            io = lax.iota(jnp.int32, 16)
            msk = v >= tv
            mi = jnp.where(msk, 1, 0).astype(jnp.int32)
            s_ = mi
            for sh in (8, 4, 2, 1):
                s_ = s_ + s_[io ^ sh]
            cnt = s_[0]

            @pl.when(cnt > 0)
            def _():
                holes = 1 - mi
                hincl = holes
                for sh in (1, 2, 4, 8):
                    hincl = hincl + jnp.where(
                        io >= sh, hincl[jnp.maximum(io - sh, 0)], 0)
                live = mi
                vv = v
                ii = io + j * 16
                r_ = jnp.where(msk, hincl - holes, 0)
                for s2 in (1, 2, 4, 8):
                    src = jnp.minimum(io + s2, 15)
                    inb = (io + s2) <= 15
                    cond = inb & (live[src] > 0) & ((r_[src] & s2) != 0)
                    moving = (live > 0) & ((r_ & s2) != 0)
                    vv = jnp.where(cond, vv[src], vv)
                    ii = jnp.where(cond, ii[src], ii)
                    r_ = jnp.where(cond, r_[src] & (~s2), r_)
                    live = jnp.where(cond, 1, jnp.where(moving, 0, live))
                cvbuf[pl.ds(off, 16)] = vv
                cibuf[pl.ds(off, 16)] = ii

            return jnp.minimum(off + cnt, CAP)

        off = lax.fori_loop(0, V // 16, chunk, 0)
        cntbuf[...] = jnp.full((16,), off, jnp.int32)
        pltpu.sync_copy(cvbuf.at[pl.ds(0, CAP)], cv_hbm.at[r])
        pltpu.sync_copy(cibuf.at[pl.ds(0, CAP)], ci_hbm.at[r])
        pltpu.sync_copy(cntbuf, cnt_hbm.at[r])


def _k2_call(logits, thr_b):
    mesh = plsc.VectorSubcoreMesh(core_axis_name="c", subcore_axis_name="s")
    fn = pl.kernel(
        _k2_body,
        mesh=mesh,
        out_type=[
            jax.ShapeDtypeStruct((B, CAP), jnp.float32),
            jax.ShapeDtypeStruct((B, CAP), jnp.int32),
            jax.ShapeDtypeStruct((B, 16), jnp.int32),
        ],
        scratch_types=[
            pltpu.VMEM((V,), jnp.float32),
            pltpu.VMEM((CAP + 16,), jnp.float32),
            pltpu.VMEM((CAP + 16,), jnp.int32),
            pltpu.VMEM((16,), jnp.float32),
            pltpu.VMEM((16,), jnp.int32),
        ],
    )
    return fn(logits, thr_b)


def _partner(a, j, ishi):
    return jnp.where(ishi, jnp.roll(a, j, axis=1), jnp.roll(a, -j, axis=1))


def _k3_body(cv_ref, ci_ref, cnt_ref, m_ref, z_ref, tp_ref, mp_ref, tk_ref,
             seed_ref, pos_ref, tok_ref, tbv_ref, tbi_ref):
    lane = lax.broadcasted_iota(jnp.int32, (B, CAP), 1)
    cnt = cnt_ref[...][:, :1]
    valid = lane < jnp.minimum(cnt, CAP)
    v = jnp.where(valid, cv_ref[...], -jnp.inf)
    ix = jnp.where(valid, ci_ref[...], -1)

    kk = 2
    while kk <= CAP:
        j = kk // 2
        while j >= 1:
            ishi = (lane & j) != 0
            pv = _partner(v, j, ishi)
            pi = _partner(ix, j, ishi)
            greater = (v > pv) | ((v == pv) & (ix > pi))
            block_asc = (lane & kk) != 0
            keep_max = ishi == block_asc
            maxv = jnp.where(greater, v, pv)
            maxi = jnp.where(greater, ix, pi)
            minv = jnp.where(greater, pv, v)
            mini = jnp.where(greater, pi, ix)
            v = jnp.where(keep_max, maxv, minv)
            ix = jnp.where(keep_max, maxi, mini)
            j //= 2
        kk *= 2

    m = m_ref[...]
    z = z_ref[...]
    p_sort = jnp.where(v == -jnp.inf, 0.0, jnp.exp(v - m) / z)

    cs = p_sort
    sft = 1
    while sft < CAP:
        cs = cs + jnp.where(lane >= sft, jnp.roll(cs, sft, axis=1), 0.0)
        sft *= 2

    ps = jnp.where(lane >= tk_ref[...], 0.0, p_sort)
    ps = jnp.where(cs - ps > tp_ref[...], 0.0, ps)
    minp_thr = ps[:, :1] * mp_ref[...]
    ps = jnp.where(ps < minp_thr, 0.0, ps)

    col = lane.astype(jnp.uint32)
    seed = seed_ref[...].astype(jnp.uint32)
    pos = pos_ref[...].astype(jnp.uint32)
    step_seed = (seed * jnp.uint32(19349663)) ^ (pos * jnp.uint32(73856093))
    hashed = (step_seed * jnp.uint32(8589934591 % (2 ** 32))) ^ (
        col * jnp.uint32(479001599))
    u = (hashed % jnp.uint32(2 ** 24)).astype(jnp.float32) / float(2 ** 24)
    u = jnp.clip(u, EPS, 1.0 - EPS)
    gumbel = -jnp.log(-jnp.log(u))
    perturbed = jnp.log(ps + EPS) + gumbel

    pmax = jnp.max(perturbed, axis=1, keepdims=True)
    s_rank = jnp.min(jnp.where(perturbed == pmax, lane, CAP), axis=1,
                     keepdims=True)
    tok_ref[...] = jnp.sum(jnp.where(lane == s_rank, ix, 0), axis=1,
                           keepdims=True)

    n = jnp.sum((ps > 0.0).astype(jnp.int32), axis=1, keepdims=True)
    sel = lane == (n - 1)
    tbv_ref[...] = jnp.sum(jnp.where(sel, v, 0.0), axis=1, keepdims=True)
    tbi_ref[...] = jnp.sum(jnp.where(sel, ix, 0), axis=1, keepdims=True)


def _k3_call(cv, ci, cnts, m, z, top_ps, min_ps, top_ks, seed, pos):
    return pl.pallas_call(
        _k3_body,
        out_shape=[
            jax.ShapeDtypeStruct((B, 1), jnp.int32),
            jax.ShapeDtypeStruct((B, 1), jnp.float32),
            jax.ShapeDtypeStruct((B, 1), jnp.int32),
        ],
    )(cv, ci, cnts, m, z, top_ps, min_ps, top_ks, seed, pos)


def _k4_body(x_ref, m_ref, z_ref, tbv_ref, tbi_ref, out_ref):
    x = x_ref[...]
    col = lax.broadcasted_iota(jnp.int32, (ROW_BLK, V), 1)
    keep = (x > tbv_ref[...]) | ((x == tbv_ref[...]) & (col >= tbi_ref[...]))
    out_ref[...] = jnp.where(keep, jnp.exp(x - m_ref[...]) / z_ref[...], 0.0)


def _k4_call(logits, m, z, tbv, tbi):
    row_spec = pl.BlockSpec((ROW_BLK, 1), lambda i: (i, 0))
    return pl.pallas_call(
        _k4_body,
        grid=(N_BLK,),
        in_specs=[pl.BlockSpec((ROW_BLK, V), lambda i: (i, 0)),
                  row_spec, row_spec, row_spec, row_spec],
        out_specs=pl.BlockSpec((ROW_BLK, V), lambda i: (i, 0)),
        out_shape=jax.ShapeDtypeStruct((B, V), jnp.float32),
    )(logits, m, z, tbv, tbi)


@jax.jit
def kernel(logits, top_ps, min_ps, top_ks, sampling_seed, positions):
    m, z, thr = _k1_call(logits)
    thr_b = jnp.broadcast_to(thr, (B, 16))
    cv, ci, cnts = _k2_call(logits, thr_b)
    tok, tbv, tbi = _k3_call(
        cv, ci, cnts, m, z,
        top_ps.reshape(B, 1), min_ps.reshape(B, 1),
        top_ks.reshape(B, 1), sampling_seed.reshape(B, 1),
        positions.reshape(B, 1))
    fp = _k4_call(logits, m, z, tbv, tbi)
    return tok[:, 0].astype(jnp.int32), fp


# speedup vs baseline: 47.2000x; 1.0954x over previous
import functools

import jax
import jax.numpy as jnp
from jax import lax
from jax.experimental import pallas as pl
from jax.experimental.pallas import tpu as pltpu
from jax.experimental.pallas import tpu_sc as plsc

B = 128
V = 100000
CAP = 2048
KMAX = 1024
EPS = 1e-10
ROW_BLK = 16
N_BLK = B // ROW_BLK


def _k1_body(x_ref, m_ref, z_ref, thr_ref):
    x = x_ref[...]
    m = jnp.max(x, axis=1, keepdims=True)
    z = jnp.sum(jnp.exp(x - m), axis=1, keepdims=True)

    i = lax.bitcast_convert_type(x, jnp.int32)
    u = i.astype(jnp.uint32)
    u = jnp.where(i >= 0, u + jnp.uint32(0x80000000), ~u)

    def bit_step(k, t):
        bit = (jnp.uint32(31) - k.astype(jnp.uint32))
        cand = t | (jnp.uint32(1) << bit)
        cnt = jnp.sum((u >= cand).astype(jnp.int32), axis=1, keepdims=True)
        return jnp.where(cnt >= KMAX, cand, t)

    t = lax.fori_loop(0, 16, bit_step, jnp.zeros((ROW_BLK, 1), jnp.uint32))
    ti = jnp.where(t >= jnp.uint32(0x80000000), t - jnp.uint32(0x80000000), ~t)
    thr = lax.bitcast_convert_type(ti, jnp.float32)

    m_ref[...] = m
    z_ref[...] = z
    thr_ref[...] = thr


def _k1_call(logits):
    out = jax.ShapeDtypeStruct((B, 1), jnp.float32)
    return pl.pallas_call(
        _k1_body,
        grid=(N_BLK,),
        in_specs=[pl.BlockSpec((ROW_BLK, V), lambda i: (i, 0))],
        out_specs=[pl.BlockSpec((ROW_BLK, 1), lambda i: (i, 0))] * 3,
        out_shape=[out, out, out],
    )(logits)


def _k2_body(logits_hbm, thr_hbm, cv_hbm, ci_hbm, cnt_hbm,
             xbuf, cvbuf, cibuf, tbuf, cntbuf):
    c = lax.axis_index("c")
    s = lax.axis_index("s")
    wid = s * 2 + c
    for rr in range(4):
        r = wid * 4 + rr
        pltpu.sync_copy(thr_hbm.at[r], tbuf)
        pltpu.sync_copy(logits_hbm.at[r], xbuf)
        tv = tbuf[...]

        def chunk(j, off):
            v = xbuf[pl.ds(j * 16, 16)]
            io = lax.iota(jnp.int32, 16)
            msk = v >= tv
            mi = jnp.where(msk, 1, 0).astype(jnp.int32)
            s_ = mi
            for sh in (8, 4, 2, 1):
                s_ = s_ + s_[io ^ sh]
            cnt = s_[0]

            @pl.when(cnt > 0)
            def _():
                holes = 1 - mi
                hincl = holes
                for sh in (1, 2, 4, 8):
                    hincl = hincl + jnp.where(
                        io >= sh, hincl[jnp.maximum(io - sh, 0)], 0)
                live = mi
                vv = v
                ii = io + j * 16
                r_ = jnp.where(msk, hincl - holes, 0)
                for s2 in (1, 2, 4, 8):
                    src = jnp.minimum(io + s2, 15)
                    inb = (io + s2) <= 15
                    cond = inb & (live[src] > 0) & ((r_[src] & s2) != 0)
                    moving = (live > 0) & ((r_ & s2) != 0)
                    vv = jnp.where(cond, vv[src], vv)
                    ii = jnp.where(cond, ii[src], ii)
                    r_ = jnp.where(cond, r_[src] & (~s2), r_)
                    live = jnp.where(cond, 1, jnp.where(moving, 0, live))
                cvbuf[pl.ds(off, 16)] = vv
                cibuf[pl.ds(off, 16)] = ii

            return jnp.minimum(off + cnt, CAP)

        off = lax.fori_loop(0, V // 16, chunk, 0)
        cntbuf[...] = jnp.full((16,), off, jnp.int32)
        pltpu.sync_copy(cvbuf.at[pl.ds(0, CAP)], cv_hbm.at[r])
        pltpu.sync_copy(cibuf.at[pl.ds(0, CAP)], ci_hbm.at[r])
        pltpu.sync_copy(cntbuf, cnt_hbm.at[r])


def _k2_call(logits, thr_b):
    mesh = plsc.VectorSubcoreMesh(core_axis_name="c", subcore_axis_name="s")
    fn = pl.kernel(
        _k2_body,
        mesh=mesh,
        out_type=[
            jax.ShapeDtypeStruct((B, CAP), jnp.float32),
            jax.ShapeDtypeStruct((B, CAP), jnp.int32),
            jax.ShapeDtypeStruct((B, 16), jnp.int32),
        ],
        scratch_types=[
            pltpu.VMEM((V,), jnp.float32),
            pltpu.VMEM((CAP + 16,), jnp.float32),
            pltpu.VMEM((CAP + 16,), jnp.int32),
            pltpu.VMEM((16,), jnp.float32),
            pltpu.VMEM((16,), jnp.int32),
        ],
    )
    return fn(logits, thr_b)


def _partner(a, j, ishi):
    return jnp.where(ishi, pltpu.roll(a, j, 1), pltpu.roll(a, CAP - j, 1))


def _k3_body(cv_ref, ci_ref, cnt_ref, m_ref, z_ref, tp_ref, mp_ref, tk_ref,
             seed_ref, pos_ref, tok_ref, tbv_ref, tbi_ref):
    lane = lax.broadcasted_iota(jnp.int32, (B, CAP), 1)
    cnt = cnt_ref[...][:, :1]
    valid = lane < jnp.minimum(cnt, CAP)
    v = jnp.where(valid, cv_ref[...], -jnp.inf)
    ix = jnp.where(valid, ci_ref[...], -1)

    kk = 2
    while kk <= CAP:
        j = kk // 2
        while j >= 1:
            ishi = (lane & j) != 0
            pv = _partner(v, j, ishi)
            pi = _partner(ix, j, ishi)
            greater = (v > pv) | ((v == pv) & (ix > pi))
            block_asc = (lane & kk) != 0
            keep_max = ishi == block_asc
            maxv = jnp.where(greater, v, pv)
            maxi = jnp.where(greater, ix, pi)
            minv = jnp.where(greater, pv, v)
            mini = jnp.where(greater, pi, ix)
            v = jnp.where(keep_max, maxv, minv)
            ix = jnp.where(keep_max, maxi, mini)
            j //= 2
        kk *= 2

    m = m_ref[...]
    z = z_ref[...]
    p_sort = jnp.where(v == -jnp.inf, 0.0, jnp.exp(v - m) / z)

    cs = p_sort
    sft = 1
    while sft < CAP:
        cs = cs + jnp.where(lane >= sft, pltpu.roll(cs, sft, 1), 0.0)
        sft *= 2

    ps = jnp.where(lane >= tk_ref[...], 0.0, p_sort)
    ps = jnp.where(cs - ps > tp_ref[...], 0.0, ps)
    minp_thr = ps[:, :1] * mp_ref[...]
    ps = jnp.where(ps < minp_thr, 0.0, ps)

    col = lane.astype(jnp.uint32)
    seed = seed_ref[...].astype(jnp.uint32)
    pos = pos_ref[...].astype(jnp.uint32)
    step_seed = (seed * jnp.uint32(19349663)) ^ (pos * jnp.uint32(73856093))
    hashed = (step_seed * jnp.uint32(8589934591 % (2 ** 32))) ^ (
        col * jnp.uint32(479001599))
    u = (hashed % jnp.uint32(2 ** 24)).astype(jnp.float32) / float(2 ** 24)
    u = jnp.clip(u, EPS, 1.0 - EPS)
    gumbel = -jnp.log(-jnp.log(u))
    perturbed = jnp.log(ps + EPS) + gumbel

    pmax = jnp.max(perturbed, axis=1, keepdims=True)
    s_rank = jnp.min(jnp.where(perturbed == pmax, lane, CAP), axis=1,
                     keepdims=True)
    tok_ref[...] = jnp.sum(jnp.where(lane == s_rank, ix, 0), axis=1,
                           keepdims=True)

    n = jnp.sum((ps > 0.0).astype(jnp.int32), axis=1, keepdims=True)
    sel = lane == (n - 1)
    tbv_ref[...] = jnp.sum(jnp.where(sel, v, 0.0), axis=1, keepdims=True)
    tbi_ref[...] = jnp.sum(jnp.where(sel, ix, 0), axis=1, keepdims=True)


def _k3_call(cv, ci, cnts, m, z, top_ps, min_ps, top_ks, seed, pos):
    return pl.pallas_call(
        _k3_body,
        out_shape=[
            jax.ShapeDtypeStruct((B, 1), jnp.int32),
            jax.ShapeDtypeStruct((B, 1), jnp.float32),
            jax.ShapeDtypeStruct((B, 1), jnp.int32),
        ],
    )(cv, ci, cnts, m, z, top_ps, min_ps, top_ks, seed, pos)


def _k4_body(x_ref, m_ref, z_ref, tbv_ref, tbi_ref, out_ref):
    x = x_ref[...]
    col = lax.broadcasted_iota(jnp.int32, (ROW_BLK, V), 1)
    keep = (x > tbv_ref[...]) | ((x == tbv_ref[...]) & (col >= tbi_ref[...]))
    out_ref[...] = jnp.where(keep, jnp.exp(x - m_ref[...]) / z_ref[...], 0.0)


def _k4_call(logits, m, z, tbv, tbi):
    row_spec = pl.BlockSpec((ROW_BLK, 1), lambda i: (i, 0))
    return pl.pallas_call(
        _k4_body,
        grid=(N_BLK,),
        in_specs=[pl.BlockSpec((ROW_BLK, V), lambda i: (i, 0)),
                  row_spec, row_spec, row_spec, row_spec],
        out_specs=pl.BlockSpec((ROW_BLK, V), lambda i: (i, 0)),
        out_shape=jax.ShapeDtypeStruct((B, V), jnp.float32),
    )(logits, m, z, tbv, tbi)


@jax.jit
def kernel(logits, top_ps, min_ps, top_ks, sampling_seed, positions):
    m, z, thr = _k1_call(logits)
    thr_b = jnp.broadcast_to(thr, (B, 16))
    cv, ci, cnts = _k2_call(logits, thr_b)
    tok, tbv, tbi = _k3_call(
        cv, ci, cnts, m, z,
        top_ps.reshape(B, 1), min_ps.reshape(B, 1),
        top_ks.reshape(B, 1), sampling_seed.reshape(B, 1),
        positions.reshape(B, 1))
    fp = _k4_call(logits, m, z, tbv, tbi)
    return tok[:, 0].astype(jnp.int32), fp


# speedup vs baseline: 50.2624x; 1.0649x over previous
import functools

import jax
import jax.numpy as jnp
from jax import lax
from jax.experimental import pallas as pl
from jax.experimental.pallas import tpu as pltpu
from jax.experimental.pallas import tpu_sc as plsc

B = 128
V = 100000
CAP = 2048
KMAX = 1024
EPS = 1e-10
ROW_BLK = 16
N_BLK = B // ROW_BLK


def _k1_body(x_ref, m_ref, z_ref, thr_ref):
    x = x_ref[...]
    m = jnp.max(x, axis=1, keepdims=True)
    z = jnp.sum(jnp.exp(x - m), axis=1, keepdims=True)

    i = lax.bitcast_convert_type(x, jnp.int32)
    u = i.astype(jnp.uint32)
    u = jnp.where(i >= 0, u + jnp.uint32(0x80000000), ~u)

    def bit_step(k, t):
        bit = (jnp.uint32(31) - k.astype(jnp.uint32))
        cand = t | (jnp.uint32(1) << bit)
        cnt = jnp.sum((u >= cand).astype(jnp.int32), axis=1, keepdims=True)
        return jnp.where(cnt >= KMAX, cand, t)

    t = lax.fori_loop(0, 16, bit_step, jnp.zeros((ROW_BLK, 1), jnp.uint32))
    ti = jnp.where(t >= jnp.uint32(0x80000000), t - jnp.uint32(0x80000000), ~t)
    thr = lax.bitcast_convert_type(ti, jnp.float32)

    m_ref[...] = m
    z_ref[...] = z
    thr_ref[...] = thr


def _k1_call(logits):
    out = jax.ShapeDtypeStruct((B, 1), jnp.float32)
    return pl.pallas_call(
        _k1_body,
        grid=(N_BLK,),
        in_specs=[pl.BlockSpec((ROW_BLK, V), lambda i: (i, 0))],
        out_specs=[pl.BlockSpec((ROW_BLK, 1), lambda i: (i, 0))] * 3,
        out_shape=[out, out, out],
    )(logits)


def _k2_body(logits_hbm, thr_hbm, cv_hbm, ci_hbm, cnt_hbm,
             xbuf, cvbuf, cibuf, tbuf, cntbuf):
    c = lax.axis_index("c")
    s = lax.axis_index("s")
    wid = s * 2 + c
    for rr in range(4):
        r = wid * 4 + rr
        pltpu.sync_copy(thr_hbm.at[r], tbuf)
        pltpu.sync_copy(logits_hbm.at[r], xbuf)
        tv = tbuf[...]

        def chunk(j, off):
            v = xbuf[pl.ds(j * 16, 16)]
            io = lax.iota(jnp.int32, 16)
            msk = v >= tv
            mi = jnp.where(msk, 1, 0).astype(jnp.int32)
            s_ = mi
            for sh in (8, 4, 2, 1):
                s_ = s_ + s_[io ^ sh]
            cnt = s_[0]

            @pl.when(cnt > 0)
            def _():
                holes = 1 - mi
                hincl = holes
                for sh in (1, 2, 4, 8):
                    hincl = hincl + jnp.where(
                        io >= sh, hincl[jnp.maximum(io - sh, 0)], 0)
                live = mi
                vv = v
                ii = io + j * 16
                r_ = jnp.where(msk, hincl - holes, 0)
                for s2 in (1, 2, 4, 8):
                    src = jnp.minimum(io + s2, 15)
                    inb = (io + s2) <= 15
                    cond = inb & (live[src] > 0) & ((r_[src] & s2) != 0)
                    moving = (live > 0) & ((r_ & s2) != 0)
                    vv = jnp.where(cond, vv[src], vv)
                    ii = jnp.where(cond, ii[src], ii)
                    r_ = jnp.where(cond, r_[src] & (~s2), r_)
                    live = jnp.where(cond, 1, jnp.where(moving, 0, live))
                cvbuf[pl.ds(off, 16)] = vv
                cibuf[pl.ds(off, 16)] = ii

            return jnp.minimum(off + cnt, CAP)

        off = lax.fori_loop(0, V // 16, chunk, 0, unroll=8)
        cntbuf[...] = jnp.full((16,), off, jnp.int32)
        pltpu.sync_copy(cvbuf.at[pl.ds(0, CAP)], cv_hbm.at[r])
        pltpu.sync_copy(cibuf.at[pl.ds(0, CAP)], ci_hbm.at[r])
        pltpu.sync_copy(cntbuf, cnt_hbm.at[r])


def _k2_call(logits, thr_b):
    mesh = plsc.VectorSubcoreMesh(core_axis_name="c", subcore_axis_name="s")
    fn = pl.kernel(
        _k2_body,
        mesh=mesh,
        out_type=[
            jax.ShapeDtypeStruct((B, CAP), jnp.float32),
            jax.ShapeDtypeStruct((B, CAP), jnp.int32),
            jax.ShapeDtypeStruct((B, 16), jnp.int32),
        ],
        scratch_types=[
            pltpu.VMEM((V,), jnp.float32),
            pltpu.VMEM((CAP + 16,), jnp.float32),
            pltpu.VMEM((CAP + 16,), jnp.int32),
            pltpu.VMEM((16,), jnp.float32),
            pltpu.VMEM((16,), jnp.int32),
        ],
    )
    return fn(logits, thr_b)


def _partner(a, j, ishi):
    return jnp.where(ishi, pltpu.roll(a, j, 1), pltpu.roll(a, CAP - j, 1))


def _k3_body(cv_ref, ci_ref, cnt_ref, m_ref, z_ref, tp_ref, mp_ref, tk_ref,
             seed_ref, pos_ref, tok_ref, tbv_ref, tbi_ref):
    lane = lax.broadcasted_iota(jnp.int32, (B, CAP), 1)
    cnt = cnt_ref[...][:, :1]
    valid = lane < jnp.minimum(cnt, CAP)
    v = jnp.where(valid, cv_ref[...], -jnp.inf)
    ix = jnp.where(valid, ci_ref[...], -1)

    kk = 2
    while kk <= CAP:
        j = kk // 2
        while j >= 1:
            ishi = (lane & j) != 0
            pv = _partner(v, j, ishi)
            pi = _partner(ix, j, ishi)
            greater = (v > pv) | ((v == pv) & (ix > pi))
            block_asc = (lane & kk) != 0
            keep_max = ishi == block_asc
            maxv = jnp.where(greater, v, pv)
            maxi = jnp.where(greater, ix, pi)
            minv = jnp.where(greater, pv, v)
            mini = jnp.where(greater, pi, ix)
            v = jnp.where(keep_max, maxv, minv)
            ix = jnp.where(keep_max, maxi, mini)
            j //= 2
        kk *= 2

    m = m_ref[...]
    z = z_ref[...]
    p_sort = jnp.where(v == -jnp.inf, 0.0, jnp.exp(v - m) / z)

    cs = p_sort
    sft = 1
    while sft < CAP:
        cs = cs + jnp.where(lane >= sft, pltpu.roll(cs, sft, 1), 0.0)
        sft *= 2

    ps = jnp.where(lane >= tk_ref[...], 0.0, p_sort)
    ps = jnp.where(cs - ps > tp_ref[...], 0.0, ps)
    minp_thr = ps[:, :1] * mp_ref[...]
    ps = jnp.where(ps < minp_thr, 0.0, ps)

    col = lane.astype(jnp.uint32)
    seed = seed_ref[...].astype(jnp.uint32)
    pos = pos_ref[...].astype(jnp.uint32)
    step_seed = (seed * jnp.uint32(19349663)) ^ (pos * jnp.uint32(73856093))
    hashed = (step_seed * jnp.uint32(8589934591 % (2 ** 32))) ^ (
        col * jnp.uint32(479001599))
    u = (hashed % jnp.uint32(2 ** 24)).astype(jnp.float32) / float(2 ** 24)
    u = jnp.clip(u, EPS, 1.0 - EPS)
    gumbel = -jnp.log(-jnp.log(u))
    perturbed = jnp.log(ps + EPS) + gumbel

    pmax = jnp.max(perturbed, axis=1, keepdims=True)
    s_rank = jnp.min(jnp.where(perturbed == pmax, lane, CAP), axis=1,
                     keepdims=True)
    tok_ref[...] = jnp.sum(jnp.where(lane == s_rank, ix, 0), axis=1,
                           keepdims=True)

    n = jnp.sum((ps > 0.0).astype(jnp.int32), axis=1, keepdims=True)
    sel = lane == (n - 1)
    tbv_ref[...] = jnp.sum(jnp.where(sel, v, 0.0), axis=1, keepdims=True)
    tbi_ref[...] = jnp.sum(jnp.where(sel, ix, 0), axis=1, keepdims=True)


def _k3_call(cv, ci, cnts, m, z, top_ps, min_ps, top_ks, seed, pos):
    return pl.pallas_call(
        _k3_body,
        out_shape=[
            jax.ShapeDtypeStruct((B, 1), jnp.int32),
            jax.ShapeDtypeStruct((B, 1), jnp.float32),
            jax.ShapeDtypeStruct((B, 1), jnp.int32),
        ],
    )(cv, ci, cnts, m, z, top_ps, min_ps, top_ks, seed, pos)


def _k4_body(x_ref, m_ref, z_ref, tbv_ref, tbi_ref, out_ref):
    x = x_ref[...]
    col = lax.broadcasted_iota(jnp.int32, (ROW_BLK, V), 1)
    keep = (x > tbv_ref[...]) | ((x == tbv_ref[...]) & (col >= tbi_ref[...]))
    out_ref[...] = jnp.where(keep, jnp.exp(x - m_ref[...]) / z_ref[...], 0.0)


def _k4_call(logits, m, z, tbv, tbi):
    row_spec = pl.BlockSpec((ROW_BLK, 1), lambda i: (i, 0))
    return pl.pallas_call(
        _k4_body,
        grid=(N_BLK,),
        in_specs=[pl.BlockSpec((ROW_BLK, V), lambda i: (i, 0)),
                  row_spec, row_spec, row_spec, row_spec],
        out_specs=pl.BlockSpec((ROW_BLK, V), lambda i: (i, 0)),
        out_shape=jax.ShapeDtypeStruct((B, V), jnp.float32),
    )(logits, m, z, tbv, tbi)


@jax.jit
def kernel(logits, top_ps, min_ps, top_ks, sampling_seed, positions):
    m, z, thr = _k1_call(logits)
    thr_b = jnp.broadcast_to(thr, (B, 16))
    cv, ci, cnts = _k2_call(logits, thr_b)
    tok, tbv, tbi = _k3_call(
        cv, ci, cnts, m, z,
        top_ps.reshape(B, 1), min_ps.reshape(B, 1),
        top_ks.reshape(B, 1), sampling_seed.reshape(B, 1),
        positions.reshape(B, 1))
    fp = _k4_call(logits, m, z, tbv, tbi)
    return tok[:, 0].astype(jnp.int32), fp


# speedup vs baseline: 59.6107x; 1.1860x over previous
import functools

import jax
import jax.numpy as jnp
from jax import lax
from jax.experimental import pallas as pl
from jax.experimental.pallas import tpu as pltpu
from jax.experimental.pallas import tpu_sc as plsc

B = 128
V = 100000
CAP = 2048
KMAX = 1024
EPS = 1e-10
ROW_BLK = 16
N_BLK = B // ROW_BLK


def _k1_body(x_ref, m_ref, z_ref, thr_ref):
    x = x_ref[...]
    m = jnp.max(x, axis=1, keepdims=True)
    z = jnp.sum(jnp.exp(x - m), axis=1, keepdims=True)

    i = lax.bitcast_convert_type(x, jnp.int32)
    u = i.astype(jnp.uint32)
    u = jnp.where(i >= 0, u + jnp.uint32(0x80000000), ~u)

    def bit_step(k, t):
        bit = (jnp.uint32(31) - k.astype(jnp.uint32))
        cand = t | (jnp.uint32(1) << bit)
        cnt = jnp.sum((u >= cand).astype(jnp.int32), axis=1, keepdims=True)
        return jnp.where(cnt >= KMAX, cand, t)

    t = lax.fori_loop(0, 16, bit_step, jnp.zeros((ROW_BLK, 1), jnp.uint32))
    ti = jnp.where(t >= jnp.uint32(0x80000000), t - jnp.uint32(0x80000000), ~t)
    thr = lax.bitcast_convert_type(ti, jnp.float32)

    m_ref[...] = m
    z_ref[...] = z
    thr_ref[...] = thr


def _k1_call(logits):
    out = jax.ShapeDtypeStruct((B, 1), jnp.float32)
    return pl.pallas_call(
        _k1_body,
        grid=(N_BLK,),
        in_specs=[pl.BlockSpec((ROW_BLK, V), lambda i: (i, 0))],
        out_specs=[pl.BlockSpec((ROW_BLK, 1), lambda i: (i, 0))] * 3,
        out_shape=[out, out, out],
    )(logits)


def _k2_body(logits_hbm, thr_hbm, cv_hbm, ci_hbm, cnt_hbm,
             xbuf, cvbuf, cibuf, tbuf, cntbuf):
    c = lax.axis_index("c")
    s = lax.axis_index("s")
    wid = s * 2 + c
    for rr in range(4):
        r = wid * 4 + rr
        pltpu.sync_copy(thr_hbm.at[r], tbuf)
        pltpu.sync_copy(logits_hbm.at[r], xbuf)
        tv = tbuf[...]

        io = lax.iota(jnp.int32, 16)

        def compact16(v, msk, mi, ii0, off):
            holes = 1 - mi
            hincl = holes
            for sh in (1, 2, 4, 8):
                hincl = hincl + jnp.where(
                    io >= sh, hincl[jnp.maximum(io - sh, 0)], 0)
            cnt = 16 - hincl[15]

            def do():
                live = mi
                vv = v
                ii = ii0
                r_ = jnp.where(msk, hincl - holes, 0)
                for s2 in (1, 2, 4, 8):
                    src = jnp.minimum(io + s2, 15)
                    inb = (io + s2) <= 15
                    cond = inb & (live[src] > 0) & ((r_[src] & s2) != 0)
                    moving = (live > 0) & ((r_ & s2) != 0)
                    vv = jnp.where(cond, vv[src], vv)
                    ii = jnp.where(cond, ii[src], ii)
                    r_ = jnp.where(cond, r_[src] & (~s2), r_)
                    live = jnp.where(cond, 1, jnp.where(moving, 0, live))
                cvbuf[pl.ds(off, 16)] = vv
                cibuf[pl.ds(off, 16)] = ii
                return jnp.minimum(off + cnt, CAP)

            return lax.cond(cnt > 0, do, lambda: off)

        def group(g, off):
            b = g * 64
            vs = [xbuf[pl.ds(b + 16 * i, 16)] for i in range(4)]
            msks = [v >= tv for v in vs]
            mis = [jnp.where(m, 1, 0).astype(jnp.int32) for m in msks]
            tot = mis[0] + mis[1] + mis[2] + mis[3]
            s_ = tot
            for sh in (8, 4, 2, 1):
                s_ = s_ + s_[io ^ sh]

            def do_group():
                o = off
                for i in range(4):
                    o = compact16(vs[i], msks[i], mis[i], io + b + 16 * i, o)
                return o

            return lax.cond(s_[0] > 0, do_group, lambda: off)

        off = lax.fori_loop(0, V // 64, group, 0, unroll=2)
        for tb in range((V // 64) * 64, V, 16):
            vt = xbuf[pl.ds(tb, 16)]
            mt = vt >= tv
            off = compact16(vt, mt, jnp.where(mt, 1, 0).astype(jnp.int32),
                            io + tb, off)
        cntbuf[...] = jnp.full((16,), off, jnp.int32)
        pltpu.sync_copy(cvbuf.at[pl.ds(0, CAP)], cv_hbm.at[r])
        pltpu.sync_copy(cibuf.at[pl.ds(0, CAP)], ci_hbm.at[r])
        pltpu.sync_copy(cntbuf, cnt_hbm.at[r])


def _k2_call(logits, thr_b):
    mesh = plsc.VectorSubcoreMesh(core_axis_name="c", subcore_axis_name="s")
    fn = pl.kernel(
        _k2_body,
        mesh=mesh,
        out_type=[
            jax.ShapeDtypeStruct((B, CAP), jnp.float32),
            jax.ShapeDtypeStruct((B, CAP), jnp.int32),
            jax.ShapeDtypeStruct((B, 16), jnp.int32),
        ],
        scratch_types=[
            pltpu.VMEM((V,), jnp.float32),
            pltpu.VMEM((CAP + 16,), jnp.float32),
            pltpu.VMEM((CAP + 16,), jnp.int32),
            pltpu.VMEM((16,), jnp.float32),
            pltpu.VMEM((16,), jnp.int32),
        ],
    )
    return fn(logits, thr_b)


def _partner(a, j, ishi):
    return jnp.where(ishi, pltpu.roll(a, j, 1), pltpu.roll(a, CAP - j, 1))


def _k3_body(cv_ref, ci_ref, cnt_ref, m_ref, z_ref, tp_ref, mp_ref, tk_ref,
             seed_ref, pos_ref, tok_ref, tbv_ref, tbi_ref):
    lane = lax.broadcasted_iota(jnp.int32, (B, CAP), 1)
    cnt = cnt_ref[...][:, :1]
    valid = lane < jnp.minimum(cnt, CAP)
    v = jnp.where(valid, cv_ref[...], -jnp.inf)
    ix = jnp.where(valid, ci_ref[...], -1)

    kk = 2
    while kk <= CAP:
        j = kk // 2
        while j >= 1:
            ishi = (lane & j) != 0
            pv = _partner(v, j, ishi)
            pi = _partner(ix, j, ishi)
            greater = (v > pv) | ((v == pv) & (ix > pi))
            block_asc = (lane & kk) != 0
            keep_max = ishi == block_asc
            maxv = jnp.where(greater, v, pv)
            maxi = jnp.where(greater, ix, pi)
            minv = jnp.where(greater, pv, v)
            mini = jnp.where(greater, pi, ix)
            v = jnp.where(keep_max, maxv, minv)
            ix = jnp.where(keep_max, maxi, mini)
            j //= 2
        kk *= 2

    m = m_ref[...]
    z = z_ref[...]
    p_sort = jnp.where(v == -jnp.inf, 0.0, jnp.exp(v - m) / z)

    cs = p_sort
    sft = 1
    while sft < CAP:
        cs = cs + jnp.where(lane >= sft, pltpu.roll(cs, sft, 1), 0.0)
        sft *= 2

    ps = jnp.where(lane >= tk_ref[...], 0.0, p_sort)
    ps = jnp.where(cs - ps > tp_ref[...], 0.0, ps)
    minp_thr = ps[:, :1] * mp_ref[...]
    ps = jnp.where(ps < minp_thr, 0.0, ps)

    col = lane.astype(jnp.uint32)
    seed = seed_ref[...].astype(jnp.uint32)
    pos = pos_ref[...].astype(jnp.uint32)
    step_seed = (seed * jnp.uint32(19349663)) ^ (pos * jnp.uint32(73856093))
    hashed = (step_seed * jnp.uint32(8589934591 % (2 ** 32))) ^ (
        col * jnp.uint32(479001599))
    u = (hashed % jnp.uint32(2 ** 24)).astype(jnp.float32) / float(2 ** 24)
    u = jnp.clip(u, EPS, 1.0 - EPS)
    gumbel = -jnp.log(-jnp.log(u))
    perturbed = jnp.log(ps + EPS) + gumbel

    pmax = jnp.max(perturbed, axis=1, keepdims=True)
    s_rank = jnp.min(jnp.where(perturbed == pmax, lane, CAP), axis=1,
                     keepdims=True)
    tok_ref[...] = jnp.sum(jnp.where(lane == s_rank, ix, 0), axis=1,
                           keepdims=True)

    n = jnp.sum((ps > 0.0).astype(jnp.int32), axis=1, keepdims=True)
    sel = lane == (n - 1)
    tbv_ref[...] = jnp.sum(jnp.where(sel, v, 0.0), axis=1, keepdims=True)
    tbi_ref[...] = jnp.sum(jnp.where(sel, ix, 0), axis=1, keepdims=True)


def _k3_call(cv, ci, cnts, m, z, top_ps, min_ps, top_ks, seed, pos):
    return pl.pallas_call(
        _k3_body,
        out_shape=[
            jax.ShapeDtypeStruct((B, 1), jnp.int32),
            jax.ShapeDtypeStruct((B, 1), jnp.float32),
            jax.ShapeDtypeStruct((B, 1), jnp.int32),
        ],
    )(cv, ci, cnts, m, z, top_ps, min_ps, top_ks, seed, pos)


def _k4_body(x_ref, m_ref, z_ref, tbv_ref, tbi_ref, out_ref):
    x = x_ref[...]
    col = lax.broadcasted_iota(jnp.int32, (ROW_BLK, V), 1)
    keep = (x > tbv_ref[...]) | ((x == tbv_ref[...]) & (col >= tbi_ref[...]))
    out_ref[...] = jnp.where(keep, jnp.exp(x - m_ref[...]) / z_ref[...], 0.0)


def _k4_call(logits, m, z, tbv, tbi):
    row_spec = pl.BlockSpec((ROW_BLK, 1), lambda i: (i, 0))
    return pl.pallas_call(
        _k4_body,
        grid=(N_BLK,),
        in_specs=[pl.BlockSpec((ROW_BLK, V), lambda i: (i, 0)),
                  row_spec, row_spec, row_spec, row_spec],
        out_specs=pl.BlockSpec((ROW_BLK, V), lambda i: (i, 0)),
        out_shape=jax.ShapeDtypeStruct((B, V), jnp.float32),
    )(logits, m, z, tbv, tbi)


@jax.jit
def kernel(logits, top_ps, min_ps, top_ks, sampling_seed, positions):
    m, z, thr = _k1_call(logits)
    thr_b = jnp.broadcast_to(thr, (B, 16))
    cv, ci, cnts = _k2_call(logits, thr_b)
    tok, tbv, tbi = _k3_call(
        cv, ci, cnts, m, z,
        top_ps.reshape(B, 1), min_ps.reshape(B, 1),
        top_ks.reshape(B, 1), sampling_seed.reshape(B, 1),
        positions.reshape(B, 1))
    fp = _k4_call(logits, m, z, tbv, tbi)
    return tok[:, 0].astype(jnp.int32), fp
